# baseline (speedup 1.0000x reference)
import dataclasses

import jax
import jax.numpy as jnp
from jax import lax
from jax.experimental import pallas as pl
from jax.experimental.pallas import tpu as pltpu
from jax.experimental.pallas import tpu_sc as plsc

EMBED_DIM = 32
CHUNK = 1024
QUARTER = CHUNK // 4
LANES = 16


def _sc_compiler_params():
    cp = pltpu.CompilerParams(use_tc_tiling_on_sc=False)
    if "needs_layout_passes" in pltpu.CompilerParams.__dataclass_fields__:
        cp = dataclasses.replace(cp, needs_layout_passes=False)
    return cp


def _gather_sc(table, tok_tiles, hist, batch):
    num_indices = hist * batch
    spc = batch // CHUNK
    mesh = plsc.VectorSubcoreMesh(core_axis_name="core", subcore_axis_name="subcore")

    @pl.kernel(
        out_type=jax.ShapeDtypeStruct((num_indices, EMBED_DIM), table.dtype),
        mesh=mesh,
        scratch_types=[pltpu.VMEM((CHUNK // 128, 128), jnp.int32)],
        compiler_params=_sc_compiler_params(),
    )
    def kern(x_hbm, i_hbm, o_hbm, perm_ref):
        def body(i_vmem, o_vmem):
            lane = lax.iota(jnp.int32, LANES)
            zero = jnp.zeros((LANES,), jnp.int32)
            base = (lane % 4) * QUARTER + lane // 4
            for v in range(CHUNK // LANES):
                src = base + 4 * v
                g = plsc.load_gather(i_vmem, [zero, src // 128, zero, src % 128])
                perm_ref[(v * LANES) // 128, pl.ds((v * LANES) % 128, LANES)] = g
            for t in range(CHUNK // 128):
                pltpu.sync_copy(
                    x_hbm.at[perm_ref.at[t]],
                    o_vmem.at[pl.ds(128 * t, 128)],
                )

        pltpu.emit_pipeline(
            body,
            grid=(num_indices // CHUNK,),
            in_specs=[
                pl.BlockSpec(
                    (1, CHUNK // 128, 1, 128),
                    index_map=lambda i: ((i // spc) // 8, i % spc, (i // spc) % 8, 0),
                )
            ],
            out_specs=[pl.BlockSpec((CHUNK, EMBED_DIM), index_map=lambda i: (i, 0))],
            core_axis_name=("core", "subcore"),
            dimension_semantics=(pltpu.PARALLEL,),
        )(i_hbm, o_hbm)

    return kern(table, tok_tiles)


def _retile_tc(out_lin, batch, hist):
    nchunk = batch // CHUNK
    x4 = out_lin.reshape(hist, nchunk, QUARTER, 128)

    def body(x_ref, o_ref):
        x = x_ref[0, 0]
        y = x.T.reshape(4, EMBED_DIM, QUARTER)
        for r in range(4):
            o_ref[0, :, pl.ds(r * QUARTER, QUARTER)] = y[r]

    return pl.pallas_call(
        body,
        grid=(hist, nchunk),
        in_specs=[pl.BlockSpec((1, 1, QUARTER, 128), lambda h, s: (h, s, 0, 0))],
        out_specs=pl.BlockSpec((1, EMBED_DIM, CHUNK), lambda h, s: (h, 0, s)),
        out_shape=jax.ShapeDtypeStruct((hist, EMBED_DIM, batch), jnp.float32),
    )(x4)


def kernel(tokens, table):
    batch, hist = tokens.shape
    tt = jnp.transpose(tokens).astype(jnp.int32)
    tok_tiles = jnp.transpose(
        tt.reshape(hist // 8, 8, batch // 128, 128), (0, 2, 1, 3)
    )
    out_lin = _gather_sc(table, tok_tiles, hist, batch)
    out_t = _retile_tc(out_lin, batch, hist)
    return jnp.transpose(out_t, (2, 0, 1))


# speedup vs baseline: 2.1696x; 2.1696x over previous
import dataclasses

import jax
import jax.numpy as jnp
from jax import lax
from jax.experimental import pallas as pl
from jax.experimental.pallas import tpu as pltpu
from jax.experimental.pallas import tpu_sc as plsc

EMBED_DIM = 32
CHUNK = 1024
QUARTER = CHUNK // 4
LANES = 16


def _sc_compiler_params():
    cp = pltpu.CompilerParams(use_tc_tiling_on_sc=False)
    if "needs_layout_passes" in pltpu.CompilerParams.__dataclass_fields__:
        cp = dataclasses.replace(cp, needs_layout_passes=False)
    return cp


def _gather_sc(table, tok_tiles, hist, batch):
    num_indices = hist * batch
    spc = batch // CHUNK
    mesh = plsc.VectorSubcoreMesh(core_axis_name="core", subcore_axis_name="subcore")

    @pl.kernel(
        out_type=jax.ShapeDtypeStruct((num_indices, EMBED_DIM), table.dtype),
        mesh=mesh,
        scratch_types=[
            pltpu.VMEM((CHUNK // 128, 128), jnp.int32),
            pltpu.SemaphoreType.DMA,
        ],
        compiler_params=_sc_compiler_params(),
    )
    def kern(x_hbm, i_hbm, o_hbm, perm_ref, sem):
        def body(i_vmem, o_vmem):
            lane = lax.iota(jnp.int32, LANES)
            zero = jnp.zeros((LANES,), jnp.int32)
            base = (lane % 4) * QUARTER + lane // 4
            copies = []
            for t in range(CHUNK // 128):
                for w in range(8):
                    v = t * 8 + w
                    src = base + 4 * v
                    g = plsc.load_gather(i_vmem, [zero, src // 128, zero, src % 128])
                    perm_ref[t, pl.ds(w * LANES, LANES)] = g
                copies.append(
                    pltpu.async_copy(
                        x_hbm.at[perm_ref.at[t]],
                        o_vmem.at[pl.ds(128 * t, 128)],
                        sem,
                    )
                )
            for c in copies:
                c.wait()

        pltpu.emit_pipeline(
            body,
            grid=(num_indices // CHUNK,),
            in_specs=[
                pl.BlockSpec(
                    (1, CHUNK // 128, 1, 128),
                    index_map=lambda i: ((i // spc) // 8, i % spc, (i // spc) % 8, 0),
                )
            ],
            out_specs=[pl.BlockSpec((CHUNK, EMBED_DIM), index_map=lambda i: (i, 0))],
            core_axis_name=("core", "subcore"),
            dimension_semantics=(pltpu.PARALLEL,),
        )(i_hbm, o_hbm)

    return kern(table, tok_tiles)


def _retile_tc(out_lin, batch, hist):
    nchunk = batch // CHUNK
    x4 = out_lin.reshape(hist, nchunk, QUARTER, 128)

    group = 4

    def body(x_ref, o_ref):
        for s2 in range(group):
            x = x_ref[0, s2]
            y = x.T.reshape(4, EMBED_DIM, QUARTER)
            for r in range(4):
                o_ref[0, :, pl.ds(s2 * CHUNK + r * QUARTER, QUARTER)] = y[r]

    return pl.pallas_call(
        body,
        grid=(hist, nchunk // group),
        in_specs=[pl.BlockSpec((1, group, QUARTER, 128), lambda h, s: (h, s, 0, 0))],
        out_specs=pl.BlockSpec((1, EMBED_DIM, group * CHUNK), lambda h, s: (h, 0, s)),
        out_shape=jax.ShapeDtypeStruct((hist, EMBED_DIM, batch), jnp.float32),
    )(x4)


def kernel(tokens, table):
    batch, hist = tokens.shape
    tt = jnp.transpose(tokens).astype(jnp.int32)
    tok_tiles = jnp.transpose(
        tt.reshape(hist // 8, 8, batch // 128, 128), (0, 2, 1, 3)
    )
    out_lin = _gather_sc(table, tok_tiles, hist, batch)
    out_t = _retile_tc(out_lin, batch, hist)
    return jnp.transpose(out_t, (2, 0, 1))


# speedup vs baseline: 2.7680x; 1.2758x over previous
import dataclasses

import jax
import jax.numpy as jnp
from jax import lax
from jax.experimental import pallas as pl
from jax.experimental.pallas import tpu as pltpu
from jax.experimental.pallas import tpu_sc as plsc

EMBED_DIM = 32
CHUNK = 1024
QUARTER = CHUNK // 4
LANES = 16


def _sc_compiler_params():
    cp = pltpu.CompilerParams(use_tc_tiling_on_sc=False)
    if "needs_layout_passes" in pltpu.CompilerParams.__dataclass_fields__:
        cp = dataclasses.replace(cp, needs_layout_passes=False)
    return cp


def _gather_sc(table, tok_tiles, hist, batch):
    num_indices = hist * batch
    spc = batch // CHUNK
    mesh = plsc.VectorSubcoreMesh(core_axis_name="core", subcore_axis_name="subcore")

    @pl.kernel(
        out_type=jax.ShapeDtypeStruct((num_indices, EMBED_DIM), table.dtype),
        mesh=mesh,
        scratch_types=[
            pltpu.VMEM((CHUNK // 128, 128), jnp.int32),
            pltpu.SemaphoreType.DMA,
        ],
        compiler_params=_sc_compiler_params(),
    )
    def kern(x_hbm, i_hbm, o_hbm, perm_ref, sem):
        def body(i_vmem, o_vmem):
            lane = lax.iota(jnp.int32, LANES)
            zero = jnp.zeros((LANES,), jnp.int32)
            base = (lane % 4) * QUARTER + lane // 4
            copies = []
            for t in range(CHUNK // 128):
                for w in range(8):
                    v = t * 8 + w
                    src = base + 4 * v
                    g = plsc.load_gather(i_vmem, [zero, src // 128, zero, src % 128])
                    perm_ref[t, pl.ds(w * LANES, LANES)] = g
                copies.append(
                    pltpu.async_copy(
                        x_hbm.at[perm_ref.at[t]],
                        o_vmem.at[pl.ds(128 * t, 128)],
                        sem,
                    )
                )
            for c in copies:
                c.wait()

        pltpu.emit_pipeline(
            body,
            grid=(num_indices // CHUNK,),
            in_specs=[
                pl.BlockSpec(
                    (1, CHUNK // 128, 1, 128),
                    index_map=lambda i: ((i // spc) // 8, i % spc, (i // spc) % 8, 0),
                )
            ],
            out_specs=[pl.BlockSpec((CHUNK, EMBED_DIM), index_map=lambda i: (i, 0))],
            core_axis_name=("core", "subcore"),
            dimension_semantics=(pltpu.PARALLEL,),
        )(i_hbm, o_hbm)

    return kern(table, tok_tiles)


def _retile_tc(out_lin, batch, hist):
    nchunk = batch // CHUNK
    x4 = out_lin.reshape(hist, nchunk, QUARTER, 128)

    group = 16

    def body(x_ref, o_ref):
        for s2 in range(group):
            x = x_ref[0, s2]
            y = x.T.reshape(4, EMBED_DIM, QUARTER)
            for r in range(4):
                o_ref[0, :, pl.ds(s2 * CHUNK + r * QUARTER, QUARTER)] = y[r]

    return pl.pallas_call(
        body,
        grid=(hist, nchunk // group),
        in_specs=[pl.BlockSpec((1, group, QUARTER, 128), lambda h, s: (h, s, 0, 0))],
        out_specs=pl.BlockSpec((1, EMBED_DIM, group * CHUNK), lambda h, s: (h, 0, s)),
        out_shape=jax.ShapeDtypeStruct((hist, EMBED_DIM, batch), jnp.float32),
    )(x4)


def kernel(tokens, table):
    batch, hist = tokens.shape
    tt = jnp.transpose(tokens).astype(jnp.int32)
    tok_tiles = jnp.transpose(
        tt.reshape(hist // 8, 8, batch // 128, 128), (0, 2, 1, 3)
    )
    out_lin = _gather_sc(table, tok_tiles, hist, batch)
    out_t = _retile_tc(out_lin, batch, hist)
    return jnp.transpose(out_t, (2, 0, 1))


# speedup vs baseline: 2.8363x; 1.0247x over previous
import dataclasses

import jax
import jax.numpy as jnp
from jax import lax
from jax.experimental import pallas as pl
from jax.experimental.pallas import tpu as pltpu
from jax.experimental.pallas import tpu_sc as plsc

EMBED_DIM = 32
CHUNK = 1024
QUARTER = CHUNK // 4
LANES = 16
NSLAB = 5


def _sc_compiler_params():
    cp = pltpu.CompilerParams(use_tc_tiling_on_sc=False)
    if "needs_layout_passes" in pltpu.CompilerParams.__dataclass_fields__:
        cp = dataclasses.replace(cp, needs_layout_passes=False)
    return cp


def _gather_sc(table, tok_tiles, hist, batch):
    num_indices = hist * batch
    spc = batch // CHUNK
    mesh = plsc.VectorSubcoreMesh(core_axis_name="core", subcore_axis_name="subcore")

    @pl.kernel(
        out_type=jax.ShapeDtypeStruct((num_indices, EMBED_DIM), table.dtype),
        mesh=mesh,
        scratch_types=[
            pltpu.VMEM((CHUNK // 128, 128), jnp.int32),
            pltpu.SemaphoreType.DMA,
        ],
        compiler_params=_sc_compiler_params(),
    )
    def kern(x_hbm, i_hbm, o_hbm, perm_ref, sem):
        def body(i_vmem, o_vmem):
            lane = lax.iota(jnp.int32, LANES)
            zero = jnp.zeros((LANES,), jnp.int32)
            base = (lane % 4) * QUARTER + lane // 4
            copies = []
            for t in range(CHUNK // 128):
                for w in range(8):
                    v = t * 8 + w
                    src = base + 4 * v
                    g = plsc.load_gather(i_vmem, [zero, src // 128, zero, src % 128])
                    perm_ref[t, pl.ds(w * LANES, LANES)] = g
                copies.append(
                    pltpu.async_copy(
                        x_hbm.at[perm_ref.at[t]],
                        o_vmem.at[pl.ds(128 * t, 128)],
                        sem,
                    )
                )
            for c in copies:
                c.wait()

        pltpu.emit_pipeline(
            body,
            grid=(num_indices // CHUNK,),
            in_specs=[
                pl.BlockSpec(
                    (1, CHUNK // 128, 1, 128),
                    index_map=lambda i: ((i // spc) // 8, i % spc, (i // spc) % 8, 0),
                )
            ],
            out_specs=[pl.BlockSpec((CHUNK, EMBED_DIM), index_map=lambda i: (i, 0))],
            core_axis_name=("core", "subcore"),
            dimension_semantics=(pltpu.PARALLEL,),
        )(i_hbm, o_hbm)

    return kern(table, tok_tiles)


def _retile_tc(out_lin_slab, acc, h0, slab_h, hist, batch):
    nchunk = batch // CHUNK
    x4 = out_lin_slab.reshape(slab_h, nchunk, QUARTER, 128)
    group = 16

    def body(x_ref, *refs):
        o_ref = refs[-1]
        for s2 in range(group):
            x = x_ref[0, s2]
            y = x.T.reshape(4, EMBED_DIM, QUARTER)
            for r in range(4):
                o_ref[0, :, pl.ds(s2 * CHUNK + r * QUARTER, QUARTER)] = y[r]

    in_specs = [pl.BlockSpec((1, group, QUARTER, 128), lambda h, s: (h, s, 0, 0))]
    operands = [x4]
    aliases = {}
    if acc is not None:
        in_specs.append(pl.BlockSpec(memory_space=pl.ANY))
        operands.append(acc)
        aliases = {1: 0}
    return pl.pallas_call(
        body,
        grid=(slab_h, nchunk // group),
        in_specs=in_specs,
        out_specs=pl.BlockSpec(
            (1, EMBED_DIM, group * CHUNK), lambda h, s: (h0 + h, 0, s)
        ),
        out_shape=jax.ShapeDtypeStruct((hist, EMBED_DIM, batch), jnp.float32),
        input_output_aliases=aliases,
    )(*operands)


def kernel(tokens, table):
    batch, hist = tokens.shape
    tt = jnp.transpose(tokens).astype(jnp.int32)
    tok_tiles = jnp.transpose(
        tt.reshape(hist // 8, 8, batch // 128, 128), (0, 2, 1, 3)
    )
    slab_h = hist // NSLAB
    slab_r = slab_h // 8
    gathered = [
        _gather_sc(
            table,
            lax.slice_in_dim(tok_tiles, k * slab_r, (k + 1) * slab_r, axis=0),
            slab_h,
            batch,
        )
        for k in range(NSLAB)
    ]
    acc = None
    for k in range(NSLAB):
        acc = _retile_tc(gathered[k], acc, k * slab_h, slab_h, hist, batch)
    return jnp.transpose(acc, (2, 0, 1))
